# Initial kernel scaffold; baseline (speedup 1.0000x reference)
import jax
import jax.numpy as jnp
from jax.experimental import pallas as pl


def kernel(x, edge_index, W1, b1, W2, b2):
    raise NotImplementedError("write your pallas kernel here")


# speedup vs baseline: 4.7702x; 4.7702x over previous
import functools

import jax
import jax.numpy as jnp
from jax import lax
from jax.experimental import pallas as pl
from jax.experimental.pallas import tpu as pltpu
from jax.experimental.pallas import tpu_sc as plsc

N = 10000
NPAD = 10240
C = 128
E = 320000
K = 3

NC = 2
NS = 16
NTILES = NC * NS
NCH = 80
NHALF = 2
NCHH = NCH // NHALF
EPT = NCH * 128
EPAD = NTILES * EPT
PAD_NODE = NPAD - 1

RBLK = 1024

_mesh = plsc.VectorSubcoreMesh(core_axis_name="c", subcore_axis_name="s")
_sc_params = pltpu.CompilerParams(needs_layout_passes=False)


@functools.partial(
    pl.kernel,
    out_type=jax.ShapeDtypeStruct((NTILES, NPAD), jnp.float32),
    mesh=_mesh,
    compiler_params=_sc_params,
    scratch_types=[
        pltpu.VMEM((NCH, 128), jnp.int32),
        pltpu.VMEM((NPAD,), jnp.float32),
    ],
)
def _sc_deg(col_hbm, out_hbm, col_v, deg_v):
    cid = lax.axis_index("c")
    sid = lax.axis_index("s")
    wid = cid * NS + sid
    pltpu.sync_copy(col_hbm.at[wid], col_v)

    zero16 = jnp.zeros((16,), jnp.float32)

    def _zero(i, carry):
        deg_v[pl.ds(i * 16, 16)] = zero16
        return carry

    lax.fori_loop(0, NPAD // 16, _zero, 0)

    one16 = jnp.ones((16,), jnp.float32)

    def _acc(j, carry):
        for k in range(8):
            idx = col_v[j, pl.ds(k * 16, 16)]
            plsc.addupdate_scatter(deg_v, [idx], one16)
        return carry

    lax.fori_loop(0, NCH, _acc, 0)
    pltpu.sync_copy(deg_v, out_hbm.at[wid])


@functools.partial(
    pl.kernel,
    out_type=jax.ShapeDtypeStruct((NC, NPAD, C), jnp.float32),
    mesh=_mesh,
    compiler_params=_sc_params,
    scratch_types=[
        pltpu.VMEM((NCHH, 128), jnp.int32),
        pltpu.VMEM((NCHH, 128), jnp.int32),
        pltpu.VMEM((2, 128, C), jnp.float32),
        pltpu.VMEM_SHARED((NPAD, C), jnp.float32),
        pltpu.SemaphoreType.DMA,
        pltpu.SemaphoreType.DMA,
    ],
)
def _sc_hop(g_hbm, row_hbm, col_hbm, out_hbm, row_v, col_v, bufs, acc_sh,
            sem0, sem1):
    cid = lax.axis_index("c")
    sid = lax.axis_index("s")
    wid = cid * NS + sid

    zero16 = jnp.zeros((16,), jnp.float32)

    def _zero(i, carry):
        for k in range(8):
            bufs[0, i, pl.ds(k * 16, 16)] = zero16
        return carry

    lax.fori_loop(0, 128, _zero, 0)
    rows_per_tile = NPAD // NS
    for t in range(rows_per_tile // 128):
        pltpu.sync_copy(bufs.at[0],
                        acc_sh.at[pl.ds(sid * rows_per_tile + t * 128, 128)])
    plsc.subcore_barrier()

    def _half(half, carry):
        pltpu.sync_copy(row_hbm.at[wid * NHALF + half], row_v)
        pltpu.sync_copy(col_hbm.at[wid * NHALF + half], col_v)

        def _body(jj, c2):
            j0 = jj * 2
            j1 = j0 + 1
            d0 = pltpu.make_async_copy(g_hbm.at[row_v.at[j0]], bufs.at[0],
                                       sem0)
            d1 = pltpu.make_async_copy(g_hbm.at[row_v.at[j1]], bufs.at[1],
                                       sem1)
            d0.start()
            d1.start()
            d0.wait()
            pltpu.sync_copy(bufs.at[0], acc_sh.at[col_v.at[j0]], add=True)
            d1.wait()
            pltpu.sync_copy(bufs.at[1], acc_sh.at[col_v.at[j1]], add=True)
            return c2

        lax.fori_loop(0, NCHH // 2, _body, 0)
        return carry

    lax.fori_loop(0, NHALF, _half, 0)
    plsc.subcore_barrier()

    for t in range(rows_per_tile // 128):
        base = sid * rows_per_tile + t * 128
        pltpu.sync_copy(acc_sh.at[pl.ds(base, 128)], bufs.at[0])
        pltpu.sync_copy(bufs.at[0], out_hbm.at[cid, pl.ds(base, 128)])


def _prep_body(dp_ref, dinv_ref, dinv2_ref):
    ones = jnp.ones((NTILES, 1), jnp.float32)
    deg = lax.dot_general(dp_ref[...], ones, (((0,), (0,)), ((), ())),
                          preferred_element_type=jnp.float32)
    safe = jnp.maximum(deg, 1.0)
    valid = deg > 0.5
    rid = lax.broadcasted_iota(jnp.int32, (NPAD, 1), 0)
    keep = valid & (rid < N)
    dinv_ref[...] = jnp.where(keep, lax.rsqrt(safe), 0.0)
    dinv2_ref[...] = jnp.where(keep, 1.0 / safe, 0.0)


def _tc_prep(degp):
    return pl.pallas_call(
        _prep_body,
        out_shape=(jax.ShapeDtypeStruct((NPAD, 1), jnp.float32),
                   jax.ShapeDtypeStruct((NPAD, 1), jnp.float32)),
    )(degp)


def _init_body(h_ref, dinv_ref, w_ref, b_ref, g_ref, acc_ref, *, relu):
    h = h_ref[...]
    if relu:
        h = jnp.maximum(h, 0.0)
    g_ref[...] = h * dinv_ref[...]
    acc_ref[...] = (jnp.dot(h, w_ref[...], preferred_element_type=jnp.float32)
                    + b_ref[...])


def _tc_init(h, dinv, w, b, relu):
    return pl.pallas_call(
        functools.partial(_init_body, relu=relu),
        grid=(NPAD // RBLK,),
        in_specs=[
            pl.BlockSpec((RBLK, C), lambda i: (i, 0)),
            pl.BlockSpec((RBLK, 1), lambda i: (i, 0)),
            pl.BlockSpec((C, C), lambda i: (0, 0)),
            pl.BlockSpec((1, C), lambda i: (0, 0)),
        ],
        out_specs=(pl.BlockSpec((RBLK, C), lambda i: (i, 0)),
                   pl.BlockSpec((RBLK, C), lambda i: (i, 0))),
        out_shape=(jax.ShapeDtypeStruct((NPAD, C), jnp.float32),
                   jax.ShapeDtypeStruct((NPAD, C), jnp.float32)),
    )(h, dinv, w, b)


def _hopc_body(p_ref, dinv_ref, dinv2_ref, w_ref, acc_ref, g_ref, accout_ref):
    t = p_ref[0] + p_ref[1]
    xs = t * dinv_ref[...]
    g_ref[...] = t * dinv2_ref[...]
    accout_ref[...] = acc_ref[...] + jnp.dot(
        xs, w_ref[...], preferred_element_type=jnp.float32)


def _tc_hopc(p, dinv, dinv2, w, acc):
    return pl.pallas_call(
        _hopc_body,
        grid=(NPAD // RBLK,),
        in_specs=[
            pl.BlockSpec((NC, RBLK, C), lambda i: (0, i, 0)),
            pl.BlockSpec((RBLK, 1), lambda i: (i, 0)),
            pl.BlockSpec((RBLK, 1), lambda i: (i, 0)),
            pl.BlockSpec((C, C), lambda i: (0, 0)),
            pl.BlockSpec((RBLK, C), lambda i: (i, 0)),
        ],
        out_specs=(pl.BlockSpec((RBLK, C), lambda i: (i, 0)),
                   pl.BlockSpec((RBLK, C), lambda i: (i, 0))),
        out_shape=(jax.ShapeDtypeStruct((NPAD, C), jnp.float32),
                   jax.ShapeDtypeStruct((NPAD, C), jnp.float32)),
    )(p, dinv, dinv2, w, acc)


def _bridge_body(p_ref, dinv_ref, w_ref, acc_ref, w0_ref, b2_ref,
                 g_ref, accout_ref):
    xs = (p_ref[0] + p_ref[1]) * dinv_ref[...]
    acc1 = acc_ref[...] + jnp.dot(xs, w_ref[...],
                                  preferred_element_type=jnp.float32)
    h2 = jnp.maximum(acc1, 0.0)
    g_ref[...] = h2 * dinv_ref[...]
    accout_ref[...] = (jnp.dot(h2, w0_ref[...],
                               preferred_element_type=jnp.float32)
                       + b2_ref[...])


def _tc_bridge(p, dinv, w, acc, w0, b2):
    return pl.pallas_call(
        _bridge_body,
        grid=(NPAD // RBLK,),
        in_specs=[
            pl.BlockSpec((NC, RBLK, C), lambda i: (0, i, 0)),
            pl.BlockSpec((RBLK, 1), lambda i: (i, 0)),
            pl.BlockSpec((C, C), lambda i: (0, 0)),
            pl.BlockSpec((RBLK, C), lambda i: (i, 0)),
            pl.BlockSpec((C, C), lambda i: (0, 0)),
            pl.BlockSpec((1, C), lambda i: (0, 0)),
        ],
        out_specs=(pl.BlockSpec((RBLK, C), lambda i: (i, 0)),
                   pl.BlockSpec((RBLK, C), lambda i: (i, 0))),
        out_shape=(jax.ShapeDtypeStruct((NPAD, C), jnp.float32),
                   jax.ShapeDtypeStruct((NPAD, C), jnp.float32)),
    )(p, dinv, w, acc, w0, b2)


def _final_body(p_ref, dinv_ref, w_ref, acc_ref, out_ref):
    xs = (p_ref[0] + p_ref[1]) * dinv_ref[...]
    z = acc_ref[...] + jnp.dot(xs, w_ref[...],
                               preferred_element_type=jnp.float32)
    m = jnp.max(z, axis=1, keepdims=True)
    s = jnp.log(jnp.sum(jnp.exp(z - m), axis=1, keepdims=True))
    out_ref[...] = z - m - s


def _tc_final(p, dinv, w, acc):
    return pl.pallas_call(
        _final_body,
        grid=(NPAD // RBLK,),
        in_specs=[
            pl.BlockSpec((NC, RBLK, C), lambda i: (0, i, 0)),
            pl.BlockSpec((RBLK, 1), lambda i: (i, 0)),
            pl.BlockSpec((C, C), lambda i: (0, 0)),
            pl.BlockSpec((RBLK, C), lambda i: (i, 0)),
        ],
        out_specs=pl.BlockSpec((RBLK, C), lambda i: (i, 0)),
        out_shape=jax.ShapeDtypeStruct((NPAD, C), jnp.float32),
    )(p, dinv, w, acc)


def kernel(x, edge_index, W1, b1, W2, b2):
    row = edge_index[0].astype(jnp.int32)
    col = edge_index[1].astype(jnp.int32)
    padlen = EPAD - E
    fill = jnp.full((padlen,), PAD_NODE, jnp.int32)
    rowp = jnp.concatenate([row, fill]).reshape(NTILES * NHALF, NCHH, 128)
    colp = jnp.concatenate([col, fill]).reshape(NTILES * NHALF, NCHH, 128)
    hp = jnp.pad(x, ((0, NPAD - N), (0, 0)))
    b1r = b1.reshape(1, C)
    b2r = b2.reshape(1, C)

    degp = _sc_deg(colp.reshape(NTILES, NCH, 128))
    dinv, dinv2 = _tc_prep(degp)

    g, acc = _tc_init(hp, dinv, W1[0:C], b1r, relu=False)
    for k in (1, 2):
        p = _sc_hop(g, rowp, colp)
        g, acc = _tc_hopc(p, dinv, dinv2, W1[C * k:C * (k + 1)], acc)
    p = _sc_hop(g, rowp, colp)
    g, acc = _tc_bridge(p, dinv, W1[3 * C:4 * C], acc, W2[0:C], b2r)
    for k in (1, 2):
        p = _sc_hop(g, rowp, colp)
        g, acc = _tc_hopc(p, dinv, dinv2, W2[C * k:C * (k + 1)], acc)
    p = _sc_hop(g, rowp, colp)
    out = _tc_final(p, dinv, W2[3 * C:4 * C], acc)
    return out[:N]


# speedup vs baseline: 12.9495x; 2.7146x over previous
import functools

import jax
import jax.numpy as jnp
from jax import lax
from jax.experimental import pallas as pl
from jax.experimental.pallas import tpu as pltpu
from jax.experimental.pallas import tpu_sc as plsc

N = 10000
NPAD = 10240
C = 128
E = 320000
K = 3

NC = 2
NS = 16
NTILES = NC * NS
NCH = 80
NHALF = 2
NCHH = NCH // NHALF
EPT = NCH * 128
EPAD = NTILES * EPT
PAD_NODE = NPAD - 1

RBLK = 1024

_mesh = plsc.VectorSubcoreMesh(core_axis_name="c", subcore_axis_name="s")
_sc_params = pltpu.CompilerParams(needs_layout_passes=False)


@functools.partial(
    pl.kernel,
    out_type=jax.ShapeDtypeStruct((NTILES, NPAD), jnp.float32),
    mesh=_mesh,
    compiler_params=_sc_params,
    scratch_types=[
        pltpu.VMEM((NCH, 128), jnp.int32),
        pltpu.VMEM((NPAD,), jnp.float32),
    ],
)
def _sc_deg(col_hbm, out_hbm, col_v, deg_v):
    cid = lax.axis_index("c")
    sid = lax.axis_index("s")
    wid = cid * NS + sid
    pltpu.sync_copy(col_hbm.at[wid], col_v)

    zero16 = jnp.zeros((16,), jnp.float32)

    def _zero(i, carry):
        deg_v[pl.ds(i * 16, 16)] = zero16
        return carry

    lax.fori_loop(0, NPAD // 16, _zero, 0)

    one16 = jnp.ones((16,), jnp.float32)

    def _acc(j, carry):
        for k in range(8):
            idx = col_v[j, pl.ds(k * 16, 16)]
            plsc.addupdate_scatter(deg_v, [idx], one16)
        return carry

    lax.fori_loop(0, NCH, _acc, 0)
    pltpu.sync_copy(deg_v, out_hbm.at[wid])


@functools.partial(
    pl.kernel,
    out_type=jax.ShapeDtypeStruct((NC, NPAD, C), jnp.float32),
    mesh=_mesh,
    compiler_params=_sc_params,
    scratch_types=[
        pltpu.VMEM((NCHH, 128), jnp.int32),
        pltpu.VMEM((NCHH, 128), jnp.int32),
        pltpu.VMEM((2, 128, C), jnp.float32),
        pltpu.VMEM_SHARED((NPAD, C), jnp.float32),
        pltpu.SemaphoreType.DMA,
        pltpu.SemaphoreType.DMA,
    ],
)
def _sc_hop(g_hbm, row_hbm, col_hbm, out_hbm, row_v, col_v, bufs, acc_sh,
            sem0, sem1):
    cid = lax.axis_index("c")
    sid = lax.axis_index("s")
    wid = cid * NS + sid

    zero16 = jnp.zeros((16,), jnp.float32)

    def _zero(i, carry):
        for k in range(8):
            bufs[0, i, pl.ds(k * 16, 16)] = zero16
        return carry

    lax.fori_loop(0, 128, _zero, 0)
    rows_per_tile = NPAD // NS
    for t in range(rows_per_tile // 128):
        pltpu.sync_copy(bufs.at[0],
                        acc_sh.at[pl.ds(sid * rows_per_tile + t * 128, 128)])
    plsc.subcore_barrier()

    def _half(half, carry):
        pltpu.sync_copy(row_hbm.at[wid * NHALF + half], row_v)
        pltpu.sync_copy(col_hbm.at[wid * NHALF + half], col_v)

        def _body(jj, c2):
            j0 = jj * 2
            j1 = j0 + 1
            d0 = pltpu.make_async_copy(g_hbm.at[row_v.at[j0]], bufs.at[0],
                                       sem0)
            d1 = pltpu.make_async_copy(g_hbm.at[row_v.at[j1]], bufs.at[1],
                                       sem1)
            d0.start()
            d1.start()
            d0.wait()
            pltpu.sync_copy(bufs.at[0], acc_sh.at[col_v.at[j0]], add=True)
            d1.wait()
            pltpu.sync_copy(bufs.at[1], acc_sh.at[col_v.at[j1]], add=True)
            return c2

        lax.fori_loop(0, NCHH // 2, _body, 0)
        return carry

    lax.fori_loop(0, NHALF, _half, 0)
    plsc.subcore_barrier()

    for t in range(rows_per_tile // 128):
        base = sid * rows_per_tile + t * 128
        pltpu.sync_copy(acc_sh.at[pl.ds(base, 128)], bufs.at[0])
        pltpu.sync_copy(bufs.at[0], out_hbm.at[cid, pl.ds(base, 128)])


def _prep_body(dp_ref, dinv_ref, dinv2_ref):
    ones = jnp.ones((NTILES, 1), jnp.float32)
    deg = lax.dot_general(dp_ref[...], ones, (((0,), (0,)), ((), ())),
                          preferred_element_type=jnp.float32)
    safe = jnp.maximum(deg, 1.0)
    valid = deg > 0.5
    rid = lax.broadcasted_iota(jnp.int32, (NPAD, 1), 0)
    keep = valid & (rid < N)
    dinv_ref[...] = jnp.where(keep, lax.rsqrt(safe), 0.0)
    dinv2_ref[...] = jnp.where(keep, 1.0 / safe, 0.0)


def _tc_prep(degp):
    return pl.pallas_call(
        _prep_body,
        out_shape=(jax.ShapeDtypeStruct((NPAD, 1), jnp.float32),
                   jax.ShapeDtypeStruct((NPAD, 1), jnp.float32)),
    )(degp)


def _init_body(h_ref, dinv_ref, w_ref, b_ref, g_ref, acc_ref, *, relu):
    h = h_ref[...]
    if relu:
        h = jnp.maximum(h, 0.0)
    g_ref[...] = h * dinv_ref[...]
    acc_ref[...] = (jnp.dot(h, w_ref[...], preferred_element_type=jnp.float32)
                    + b_ref[...])


def _tc_init(h, dinv, w, b, relu):
    return pl.pallas_call(
        functools.partial(_init_body, relu=relu),
        grid=(NPAD // RBLK,),
        in_specs=[
            pl.BlockSpec((RBLK, C), lambda i: (i, 0)),
            pl.BlockSpec((RBLK, 1), lambda i: (i, 0)),
            pl.BlockSpec((C, C), lambda i: (0, 0)),
            pl.BlockSpec((1, C), lambda i: (0, 0)),
        ],
        out_specs=(pl.BlockSpec((RBLK, C), lambda i: (i, 0)),
                   pl.BlockSpec((RBLK, C), lambda i: (i, 0))),
        out_shape=(jax.ShapeDtypeStruct((NPAD, C), jnp.float32),
                   jax.ShapeDtypeStruct((NPAD, C), jnp.float32)),
    )(h, dinv, w, b)


def _hopc_body(p_ref, dinv_ref, dinv2_ref, w_ref, acc_ref, g_ref, accout_ref):
    t = p_ref[0] + p_ref[1]
    xs = t * dinv_ref[...]
    g_ref[...] = t * dinv2_ref[...]
    accout_ref[...] = acc_ref[...] + jnp.dot(
        xs, w_ref[...], preferred_element_type=jnp.float32)


def _tc_hopc(p, dinv, dinv2, w, acc):
    return pl.pallas_call(
        _hopc_body,
        grid=(NPAD // RBLK,),
        in_specs=[
            pl.BlockSpec((NC, RBLK, C), lambda i: (0, i, 0)),
            pl.BlockSpec((RBLK, 1), lambda i: (i, 0)),
            pl.BlockSpec((RBLK, 1), lambda i: (i, 0)),
            pl.BlockSpec((C, C), lambda i: (0, 0)),
            pl.BlockSpec((RBLK, C), lambda i: (i, 0)),
        ],
        out_specs=(pl.BlockSpec((RBLK, C), lambda i: (i, 0)),
                   pl.BlockSpec((RBLK, C), lambda i: (i, 0))),
        out_shape=(jax.ShapeDtypeStruct((NPAD, C), jnp.float32),
                   jax.ShapeDtypeStruct((NPAD, C), jnp.float32)),
    )(p, dinv, dinv2, w, acc)


def _bridge_body(p_ref, dinv_ref, w_ref, acc_ref, w0_ref, b2_ref,
                 g_ref, accout_ref):
    xs = (p_ref[0] + p_ref[1]) * dinv_ref[...]
    acc1 = acc_ref[...] + jnp.dot(xs, w_ref[...],
                                  preferred_element_type=jnp.float32)
    h2 = jnp.maximum(acc1, 0.0)
    g_ref[...] = h2 * dinv_ref[...]
    accout_ref[...] = (jnp.dot(h2, w0_ref[...],
                               preferred_element_type=jnp.float32)
                       + b2_ref[...])


def _tc_bridge(p, dinv, w, acc, w0, b2):
    return pl.pallas_call(
        _bridge_body,
        grid=(NPAD // RBLK,),
        in_specs=[
            pl.BlockSpec((NC, RBLK, C), lambda i: (0, i, 0)),
            pl.BlockSpec((RBLK, 1), lambda i: (i, 0)),
            pl.BlockSpec((C, C), lambda i: (0, 0)),
            pl.BlockSpec((RBLK, C), lambda i: (i, 0)),
            pl.BlockSpec((C, C), lambda i: (0, 0)),
            pl.BlockSpec((1, C), lambda i: (0, 0)),
        ],
        out_specs=(pl.BlockSpec((RBLK, C), lambda i: (i, 0)),
                   pl.BlockSpec((RBLK, C), lambda i: (i, 0))),
        out_shape=(jax.ShapeDtypeStruct((NPAD, C), jnp.float32),
                   jax.ShapeDtypeStruct((NPAD, C), jnp.float32)),
    )(p, dinv, w, acc, w0, b2)


def _final_body(p_ref, dinv_ref, w_ref, acc_ref, out_ref):
    xs = (p_ref[0] + p_ref[1]) * dinv_ref[...]
    z = acc_ref[...] + jnp.dot(xs, w_ref[...],
                               preferred_element_type=jnp.float32)
    m = jnp.max(z, axis=1, keepdims=True)
    s = jnp.log(jnp.sum(jnp.exp(z - m), axis=1, keepdims=True))
    out_ref[...] = z - m - s


def _tc_final(p, dinv, w, acc):
    return pl.pallas_call(
        _final_body,
        grid=(NPAD // RBLK,),
        in_specs=[
            pl.BlockSpec((NC, RBLK, C), lambda i: (0, i, 0)),
            pl.BlockSpec((RBLK, 1), lambda i: (i, 0)),
            pl.BlockSpec((C, C), lambda i: (0, 0)),
            pl.BlockSpec((RBLK, C), lambda i: (i, 0)),
        ],
        out_specs=pl.BlockSpec((RBLK, C), lambda i: (i, 0)),
        out_shape=jax.ShapeDtypeStruct((NPAD, C), jnp.float32),
    )(p, dinv, w, acc)


def kernel(x, edge_index, W1, b1, W2, b2):
    row = edge_index[0].astype(jnp.int32)
    col = edge_index[1].astype(jnp.int32)
    padlen = EPAD - E
    fill = N + (jnp.arange(padlen, dtype=jnp.int32) % (NPAD - N))

    def _shard(v):
        ch = v.reshape(NCH, NTILES, 128).transpose(1, 0, 2)
        return ch.reshape(NTILES * NHALF, NCHH, 128)

    rowp = _shard(jnp.concatenate([row, fill]))
    colp = _shard(jnp.concatenate([col, fill]))
    hp = jnp.pad(x, ((0, NPAD - N), (0, 0)))
    b1r = b1.reshape(1, C)
    b2r = b2.reshape(1, C)

    degp = _sc_deg(colp.reshape(NTILES, NCH, 128))
    dinv, dinv2 = _tc_prep(degp)

    g, acc = _tc_init(hp, dinv, W1[0:C], b1r, relu=False)
    for k in (1, 2):
        p = _sc_hop(g, rowp, colp)
        g, acc = _tc_hopc(p, dinv, dinv2, W1[C * k:C * (k + 1)], acc)
    p = _sc_hop(g, rowp, colp)
    g, acc = _tc_bridge(p, dinv, W1[3 * C:4 * C], acc, W2[0:C], b2r)
    for k in (1, 2):
        p = _sc_hop(g, rowp, colp)
        g, acc = _tc_hopc(p, dinv, dinv2, W2[C * k:C * (k + 1)], acc)
    p = _sc_hop(g, rowp, colp)
    out = _tc_final(p, dinv, W2[3 * C:4 * C], acc)
    return out[:N]


# speedup vs baseline: 14.6679x; 1.1327x over previous
import functools

import jax
import jax.numpy as jnp
from jax import lax
from jax.experimental import pallas as pl
from jax.experimental.pallas import tpu as pltpu
from jax.experimental.pallas import tpu_sc as plsc

N = 10000
NPAD = 10240
C = 128
E = 320000
K = 3

NC = 2
NS = 16
NTILES = NC * NS
NCH = 80
NHALF = 2
NCHH = NCH // NHALF
EPT = NCH * 128
EPAD = NTILES * EPT
PAD_NODE = NPAD - 1

RBLK = 1024

_mesh = plsc.VectorSubcoreMesh(core_axis_name="c", subcore_axis_name="s")
_sc_params = pltpu.CompilerParams(needs_layout_passes=False)


@functools.partial(
    pl.kernel,
    out_type=jax.ShapeDtypeStruct((NTILES, NPAD), jnp.float32),
    mesh=_mesh,
    compiler_params=_sc_params,
    scratch_types=[
        pltpu.VMEM((NCH, 128), jnp.int32),
        pltpu.VMEM((NPAD,), jnp.float32),
    ],
)
def _sc_deg(col_hbm, out_hbm, col_v, deg_v):
    cid = lax.axis_index("c")
    sid = lax.axis_index("s")
    wid = cid * NS + sid
    pltpu.sync_copy(col_hbm.at[wid], col_v)

    zero16 = jnp.zeros((16,), jnp.float32)

    def _zero(i, carry):
        deg_v[pl.ds(i * 16, 16)] = zero16
        return carry

    lax.fori_loop(0, NPAD // 16, _zero, 0)

    one16 = jnp.ones((16,), jnp.float32)

    def _acc(j, carry):
        for k in range(8):
            idx = col_v[j, pl.ds(k * 16, 16)]
            plsc.addupdate_scatter(deg_v, [idx], one16)
        return carry

    lax.fori_loop(0, NCH, _acc, 0)
    pltpu.sync_copy(deg_v, out_hbm.at[wid])


@functools.partial(
    pl.kernel,
    out_type=jax.ShapeDtypeStruct((NC, NPAD, C), jnp.float32),
    mesh=_mesh,
    compiler_params=_sc_params,
    scratch_types=[
        pltpu.VMEM((NCHH, 128), jnp.int32),
        pltpu.VMEM((NCHH, 128), jnp.int32),
        pltpu.VMEM((2, 128, C), jnp.float32),
        pltpu.VMEM_SHARED((NPAD, C), jnp.float32),
        pltpu.SemaphoreType.DMA,
        pltpu.SemaphoreType.DMA,
        pltpu.SemaphoreType.DMA,
        pltpu.SemaphoreType.DMA,
    ],
)
def _sc_hop(g_hbm, row_hbm, col_hbm, out_hbm, row_v, col_v, bufs, acc_sh,
            gsem0, gsem1, ssem0, ssem1):
    cid = lax.axis_index("c")
    sid = lax.axis_index("s")
    wid = cid * NS + sid

    zero16 = jnp.zeros((16,), jnp.float32)

    def _zero(i, carry):
        for k in range(8):
            bufs[0, i, pl.ds(k * 16, 16)] = zero16
        return carry

    lax.fori_loop(0, 128, _zero, 0)
    rows_per_tile = NPAD // NS
    for t in range(rows_per_tile // 128):
        pltpu.sync_copy(bufs.at[0],
                        acc_sh.at[pl.ds(sid * rows_per_tile + t * 128, 128)])
    plsc.subcore_barrier()

    def _gat(j, slot, sem):
        return pltpu.make_async_copy(g_hbm.at[row_v.at[j]], bufs.at[slot],
                                     sem)

    def _sca(j, slot, sem):
        return pltpu.make_async_copy(bufs.at[slot], acc_sh.at[col_v.at[j]],
                                     sem)

    NB = NCHH // 2

    def _half(half, carry):
        pltpu.sync_copy(row_hbm.at[wid * NHALF + half], row_v)
        pltpu.sync_copy(col_hbm.at[wid * NHALF + half], col_v)
        _gat(0, 0, gsem0).start()

        def _body(jj, c2):
            a = jj * 2
            b = a + 1
            _gat(a, 0, gsem0).wait()

            @pl.when(jj > 0)
            def _():
                _sca(b - 2, 1, ssem1).wait()

            _gat(b, 1, gsem1).start()
            _sca(a, 0, ssem0).start(add=True)
            _gat(b, 1, gsem1).wait()
            _sca(a, 0, ssem0).wait()

            @pl.when(jj + 1 < NB)
            def _():
                _gat(a + 2, 0, gsem0).start()

            _sca(b, 1, ssem1).start(add=True)
            return c2

        lax.fori_loop(0, NB, _body, 0)
        _sca(NCHH - 1, 1, ssem1).wait()
        return carry

    lax.fori_loop(0, NHALF, _half, 0)
    plsc.subcore_barrier()

    for t in range(rows_per_tile // 128):
        base = sid * rows_per_tile + t * 128
        pltpu.sync_copy(acc_sh.at[pl.ds(base, 128)], bufs.at[0])
        pltpu.sync_copy(bufs.at[0], out_hbm.at[cid, pl.ds(base, 128)])


def _prep_body(dp_ref, dinv_ref, dinv2_ref):
    ones = jnp.ones((NTILES, 1), jnp.float32)
    deg = lax.dot_general(dp_ref[...], ones, (((0,), (0,)), ((), ())),
                          preferred_element_type=jnp.float32)
    safe = jnp.maximum(deg, 1.0)
    valid = deg > 0.5
    rid = lax.broadcasted_iota(jnp.int32, (NPAD, 1), 0)
    keep = valid & (rid < N)
    dinv_ref[...] = jnp.where(keep, lax.rsqrt(safe), 0.0)
    dinv2_ref[...] = jnp.where(keep, 1.0 / safe, 0.0)


def _tc_prep(degp):
    return pl.pallas_call(
        _prep_body,
        out_shape=(jax.ShapeDtypeStruct((NPAD, 1), jnp.float32),
                   jax.ShapeDtypeStruct((NPAD, 1), jnp.float32)),
    )(degp)


def _init_body(h_ref, dinv_ref, w_ref, b_ref, g_ref, acc_ref, *, relu):
    h = h_ref[...]
    if relu:
        h = jnp.maximum(h, 0.0)
    g_ref[...] = h * dinv_ref[...]
    acc_ref[...] = (jnp.dot(h, w_ref[...], preferred_element_type=jnp.float32)
                    + b_ref[...])


def _tc_init(h, dinv, w, b, relu):
    return pl.pallas_call(
        functools.partial(_init_body, relu=relu),
        grid=(NPAD // RBLK,),
        in_specs=[
            pl.BlockSpec((RBLK, C), lambda i: (i, 0)),
            pl.BlockSpec((RBLK, 1), lambda i: (i, 0)),
            pl.BlockSpec((C, C), lambda i: (0, 0)),
            pl.BlockSpec((1, C), lambda i: (0, 0)),
        ],
        out_specs=(pl.BlockSpec((RBLK, C), lambda i: (i, 0)),
                   pl.BlockSpec((RBLK, C), lambda i: (i, 0))),
        out_shape=(jax.ShapeDtypeStruct((NPAD, C), jnp.float32),
                   jax.ShapeDtypeStruct((NPAD, C), jnp.float32)),
    )(h, dinv, w, b)


def _hopc_body(p_ref, dinv_ref, dinv2_ref, w_ref, acc_ref, g_ref, accout_ref):
    t = p_ref[0] + p_ref[1]
    xs = t * dinv_ref[...]
    g_ref[...] = t * dinv2_ref[...]
    accout_ref[...] = acc_ref[...] + jnp.dot(
        xs, w_ref[...], preferred_element_type=jnp.float32)


def _tc_hopc(p, dinv, dinv2, w, acc):
    return pl.pallas_call(
        _hopc_body,
        grid=(NPAD // RBLK,),
        in_specs=[
            pl.BlockSpec((NC, RBLK, C), lambda i: (0, i, 0)),
            pl.BlockSpec((RBLK, 1), lambda i: (i, 0)),
            pl.BlockSpec((RBLK, 1), lambda i: (i, 0)),
            pl.BlockSpec((C, C), lambda i: (0, 0)),
            pl.BlockSpec((RBLK, C), lambda i: (i, 0)),
        ],
        out_specs=(pl.BlockSpec((RBLK, C), lambda i: (i, 0)),
                   pl.BlockSpec((RBLK, C), lambda i: (i, 0))),
        out_shape=(jax.ShapeDtypeStruct((NPAD, C), jnp.float32),
                   jax.ShapeDtypeStruct((NPAD, C), jnp.float32)),
    )(p, dinv, dinv2, w, acc)


def _bridge_body(p_ref, dinv_ref, w_ref, acc_ref, w0_ref, b2_ref,
                 g_ref, accout_ref):
    xs = (p_ref[0] + p_ref[1]) * dinv_ref[...]
    acc1 = acc_ref[...] + jnp.dot(xs, w_ref[...],
                                  preferred_element_type=jnp.float32)
    h2 = jnp.maximum(acc1, 0.0)
    g_ref[...] = h2 * dinv_ref[...]
    accout_ref[...] = (jnp.dot(h2, w0_ref[...],
                               preferred_element_type=jnp.float32)
                       + b2_ref[...])


def _tc_bridge(p, dinv, w, acc, w0, b2):
    return pl.pallas_call(
        _bridge_body,
        grid=(NPAD // RBLK,),
        in_specs=[
            pl.BlockSpec((NC, RBLK, C), lambda i: (0, i, 0)),
            pl.BlockSpec((RBLK, 1), lambda i: (i, 0)),
            pl.BlockSpec((C, C), lambda i: (0, 0)),
            pl.BlockSpec((RBLK, C), lambda i: (i, 0)),
            pl.BlockSpec((C, C), lambda i: (0, 0)),
            pl.BlockSpec((1, C), lambda i: (0, 0)),
        ],
        out_specs=(pl.BlockSpec((RBLK, C), lambda i: (i, 0)),
                   pl.BlockSpec((RBLK, C), lambda i: (i, 0))),
        out_shape=(jax.ShapeDtypeStruct((NPAD, C), jnp.float32),
                   jax.ShapeDtypeStruct((NPAD, C), jnp.float32)),
    )(p, dinv, w, acc, w0, b2)


def _final_body(p_ref, dinv_ref, w_ref, acc_ref, out_ref):
    xs = (p_ref[0] + p_ref[1]) * dinv_ref[...]
    z = acc_ref[...] + jnp.dot(xs, w_ref[...],
                               preferred_element_type=jnp.float32)
    m = jnp.max(z, axis=1, keepdims=True)
    s = jnp.log(jnp.sum(jnp.exp(z - m), axis=1, keepdims=True))
    out_ref[...] = z - m - s


def _tc_final(p, dinv, w, acc):
    return pl.pallas_call(
        _final_body,
        grid=(NPAD // RBLK,),
        in_specs=[
            pl.BlockSpec((NC, RBLK, C), lambda i: (0, i, 0)),
            pl.BlockSpec((RBLK, 1), lambda i: (i, 0)),
            pl.BlockSpec((C, C), lambda i: (0, 0)),
            pl.BlockSpec((RBLK, C), lambda i: (i, 0)),
        ],
        out_specs=pl.BlockSpec((RBLK, C), lambda i: (i, 0)),
        out_shape=jax.ShapeDtypeStruct((NPAD, C), jnp.float32),
    )(p, dinv, w, acc)


def kernel(x, edge_index, W1, b1, W2, b2):
    row = edge_index[0].astype(jnp.int32)
    col = edge_index[1].astype(jnp.int32)
    padlen = EPAD - E
    fill = N + (jnp.arange(padlen, dtype=jnp.int32) % (NPAD - N))

    def _shard(v):
        ch = v.reshape(NCH, NTILES, 128).transpose(1, 0, 2)
        return ch.reshape(NTILES * NHALF, NCHH, 128)

    rowp = _shard(jnp.concatenate([row, fill]))
    colp = _shard(jnp.concatenate([col, fill]))
    hp = jnp.pad(x, ((0, NPAD - N), (0, 0)))
    b1r = b1.reshape(1, C)
    b2r = b2.reshape(1, C)

    degp = _sc_deg(colp.reshape(NTILES, NCH, 128))
    dinv, dinv2 = _tc_prep(degp)

    g, acc = _tc_init(hp, dinv, W1[0:C], b1r, relu=False)
    for k in (1, 2):
        p = _sc_hop(g, rowp, colp)
        g, acc = _tc_hopc(p, dinv, dinv2, W1[C * k:C * (k + 1)], acc)
    p = _sc_hop(g, rowp, colp)
    g, acc = _tc_bridge(p, dinv, W1[3 * C:4 * C], acc, W2[0:C], b2r)
    for k in (1, 2):
        p = _sc_hop(g, rowp, colp)
        g, acc = _tc_hopc(p, dinv, dinv2, W2[C * k:C * (k + 1)], acc)
    p = _sc_hop(g, rowp, colp)
    out = _tc_final(p, dinv, W2[3 * C:4 * C], acc)
    return out[:N]


# speedup vs baseline: 14.8935x; 1.0154x over previous
import functools

import jax
import jax.numpy as jnp
from jax import lax
from jax.experimental import pallas as pl
from jax.experimental.pallas import tpu as pltpu
from jax.experimental.pallas import tpu_sc as plsc

N = 10000
NPAD = 10240
C = 128
E = 320000
K = 3

NC = 2
NS = 16
NTILES = NC * NS
CHUNK = 64
NCH = 160
NHALF = 4
NCHH = NCH // NHALF
NSLOT = 4
EPT = NCH * CHUNK
EPAD = NTILES * EPT
PAD_NODE = NPAD - 1

RBLK = 1024

_mesh = plsc.VectorSubcoreMesh(core_axis_name="c", subcore_axis_name="s")
_sc_params = pltpu.CompilerParams(needs_layout_passes=False)


@functools.partial(
    pl.kernel,
    out_type=jax.ShapeDtypeStruct((NTILES, NPAD), jnp.float32),
    mesh=_mesh,
    compiler_params=_sc_params,
    scratch_types=[
        pltpu.VMEM((NCH, CHUNK), jnp.int32),
        pltpu.VMEM((NPAD,), jnp.float32),
    ],
)
def _sc_deg(col_hbm, out_hbm, col_v, deg_v):
    cid = lax.axis_index("c")
    sid = lax.axis_index("s")
    wid = cid * NS + sid
    pltpu.sync_copy(col_hbm.at[wid], col_v)

    zero16 = jnp.zeros((16,), jnp.float32)

    def _zero(i, carry):
        deg_v[pl.ds(i * 16, 16)] = zero16
        return carry

    lax.fori_loop(0, NPAD // 16, _zero, 0)

    one16 = jnp.ones((16,), jnp.float32)

    def _acc(j, carry):
        for k in range(CHUNK // 16):
            idx = col_v[j, pl.ds(k * 16, 16)]
            plsc.addupdate_scatter(deg_v, [idx], one16)
        return carry

    lax.fori_loop(0, NCH, _acc, 0)
    pltpu.sync_copy(deg_v, out_hbm.at[wid])


@functools.partial(
    pl.kernel,
    out_type=jax.ShapeDtypeStruct((NC, NPAD, C), jnp.float32),
    mesh=_mesh,
    compiler_params=_sc_params,
    scratch_types=[
        pltpu.VMEM((NCHH, CHUNK), jnp.int32),
        pltpu.VMEM((NCHH, CHUNK), jnp.int32),
        pltpu.VMEM((NSLOT, CHUNK, C), jnp.float32),
        pltpu.VMEM_SHARED((NPAD, C), jnp.float32),
        pltpu.SemaphoreType.DMA,
        pltpu.SemaphoreType.DMA,
        pltpu.SemaphoreType.DMA,
        pltpu.SemaphoreType.DMA,
        pltpu.SemaphoreType.DMA,
        pltpu.SemaphoreType.DMA,
        pltpu.SemaphoreType.DMA,
        pltpu.SemaphoreType.DMA,
    ],
)
def _sc_hop(g_hbm, row_hbm, col_hbm, out_hbm, row_v, col_v, bufs, acc_sh,
            *sems):
    gsem = sems[:NSLOT]
    ssem = sems[NSLOT:]
    cid = lax.axis_index("c")
    sid = lax.axis_index("s")
    wid = cid * NS + sid

    zero16 = jnp.zeros((16,), jnp.float32)

    def _zero(i, carry):
        for k in range(8):
            bufs[0, i, pl.ds(k * 16, 16)] = zero16
        return carry

    lax.fori_loop(0, CHUNK, _zero, 0)
    rows_per_tile = NPAD // NS
    for t in range(rows_per_tile // CHUNK):
        pltpu.sync_copy(
            bufs.at[0],
            acc_sh.at[pl.ds(sid * rows_per_tile + t * CHUNK, CHUNK)])
    plsc.subcore_barrier()

    def _gat(j, slot):
        return pltpu.make_async_copy(g_hbm.at[row_v.at[j]], bufs.at[slot],
                                     gsem[slot])

    def _sca(j, slot):
        return pltpu.make_async_copy(bufs.at[slot], acc_sh.at[col_v.at[j]],
                                     ssem[slot])

    NB = NCHH // NSLOT

    def _half(half, carry):
        pltpu.sync_copy(row_hbm.at[wid * NHALF + half], row_v)
        pltpu.sync_copy(col_hbm.at[wid * NHALF + half], col_v)
        _gat(0, 0).start()
        _gat(1, 1).start()

        def _body(jj, c2):
            c0 = jj * NSLOT
            for i in range(NSLOT):
                ci = c0 + i
                s_next = (i + 2) % NSLOT
                _gat(ci, i).wait()

                @pl.when(ci >= 2)
                def _():
                    _sca(ci - 2, s_next).wait()

                @pl.when(ci + 2 < NCHH)
                def _():
                    _gat(ci + 2, s_next).start()

                _sca(ci, i).start(add=True)
            return c2

        lax.fori_loop(0, NB, _body, 0)
        _sca(NCHH - 2, (NCHH - 2) % NSLOT).wait()
        _sca(NCHH - 1, (NCHH - 1) % NSLOT).wait()
        return carry

    lax.fori_loop(0, NHALF, _half, 0)
    plsc.subcore_barrier()

    for t in range(rows_per_tile // CHUNK):
        base = sid * rows_per_tile + t * CHUNK
        pltpu.sync_copy(acc_sh.at[pl.ds(base, CHUNK)], bufs.at[0])
        pltpu.sync_copy(bufs.at[0], out_hbm.at[cid, pl.ds(base, CHUNK)])


def _prep_body(dp_ref, dinv_ref, dinv2_ref):
    ones = jnp.ones((NTILES, 1), jnp.float32)
    deg = lax.dot_general(dp_ref[...], ones, (((0,), (0,)), ((), ())),
                          preferred_element_type=jnp.float32)
    safe = jnp.maximum(deg, 1.0)
    valid = deg > 0.5
    rid = lax.broadcasted_iota(jnp.int32, (NPAD, 1), 0)
    keep = valid & (rid < N)
    dinv_ref[...] = jnp.where(keep, lax.rsqrt(safe), 0.0)
    dinv2_ref[...] = jnp.where(keep, 1.0 / safe, 0.0)


def _tc_prep(degp):
    return pl.pallas_call(
        _prep_body,
        out_shape=(jax.ShapeDtypeStruct((NPAD, 1), jnp.float32),
                   jax.ShapeDtypeStruct((NPAD, 1), jnp.float32)),
    )(degp)


def _init_body(h_ref, dinv_ref, w_ref, b_ref, g_ref, acc_ref, *, relu):
    h = h_ref[...]
    if relu:
        h = jnp.maximum(h, 0.0)
    g_ref[...] = h * dinv_ref[...]
    acc_ref[...] = (jnp.dot(h, w_ref[...], preferred_element_type=jnp.float32)
                    + b_ref[...])


def _tc_init(h, dinv, w, b, relu):
    return pl.pallas_call(
        functools.partial(_init_body, relu=relu),
        grid=(NPAD // RBLK,),
        in_specs=[
            pl.BlockSpec((RBLK, C), lambda i: (i, 0)),
            pl.BlockSpec((RBLK, 1), lambda i: (i, 0)),
            pl.BlockSpec((C, C), lambda i: (0, 0)),
            pl.BlockSpec((1, C), lambda i: (0, 0)),
        ],
        out_specs=(pl.BlockSpec((RBLK, C), lambda i: (i, 0)),
                   pl.BlockSpec((RBLK, C), lambda i: (i, 0))),
        out_shape=(jax.ShapeDtypeStruct((NPAD, C), jnp.float32),
                   jax.ShapeDtypeStruct((NPAD, C), jnp.float32)),
    )(h, dinv, w, b)


def _hopc_body(p_ref, dinv_ref, dinv2_ref, w_ref, acc_ref, g_ref, accout_ref):
    t = p_ref[0] + p_ref[1]
    xs = t * dinv_ref[...]
    g_ref[...] = t * dinv2_ref[...]
    accout_ref[...] = acc_ref[...] + jnp.dot(
        xs, w_ref[...], preferred_element_type=jnp.float32)


def _tc_hopc(p, dinv, dinv2, w, acc):
    return pl.pallas_call(
        _hopc_body,
        grid=(NPAD // RBLK,),
        in_specs=[
            pl.BlockSpec((NC, RBLK, C), lambda i: (0, i, 0)),
            pl.BlockSpec((RBLK, 1), lambda i: (i, 0)),
            pl.BlockSpec((RBLK, 1), lambda i: (i, 0)),
            pl.BlockSpec((C, C), lambda i: (0, 0)),
            pl.BlockSpec((RBLK, C), lambda i: (i, 0)),
        ],
        out_specs=(pl.BlockSpec((RBLK, C), lambda i: (i, 0)),
                   pl.BlockSpec((RBLK, C), lambda i: (i, 0))),
        out_shape=(jax.ShapeDtypeStruct((NPAD, C), jnp.float32),
                   jax.ShapeDtypeStruct((NPAD, C), jnp.float32)),
    )(p, dinv, dinv2, w, acc)


def _bridge_body(p_ref, dinv_ref, w_ref, acc_ref, w0_ref, b2_ref,
                 g_ref, accout_ref):
    xs = (p_ref[0] + p_ref[1]) * dinv_ref[...]
    acc1 = acc_ref[...] + jnp.dot(xs, w_ref[...],
                                  preferred_element_type=jnp.float32)
    h2 = jnp.maximum(acc1, 0.0)
    g_ref[...] = h2 * dinv_ref[...]
    accout_ref[...] = (jnp.dot(h2, w0_ref[...],
                               preferred_element_type=jnp.float32)
                       + b2_ref[...])


def _tc_bridge(p, dinv, w, acc, w0, b2):
    return pl.pallas_call(
        _bridge_body,
        grid=(NPAD // RBLK,),
        in_specs=[
            pl.BlockSpec((NC, RBLK, C), lambda i: (0, i, 0)),
            pl.BlockSpec((RBLK, 1), lambda i: (i, 0)),
            pl.BlockSpec((C, C), lambda i: (0, 0)),
            pl.BlockSpec((RBLK, C), lambda i: (i, 0)),
            pl.BlockSpec((C, C), lambda i: (0, 0)),
            pl.BlockSpec((1, C), lambda i: (0, 0)),
        ],
        out_specs=(pl.BlockSpec((RBLK, C), lambda i: (i, 0)),
                   pl.BlockSpec((RBLK, C), lambda i: (i, 0))),
        out_shape=(jax.ShapeDtypeStruct((NPAD, C), jnp.float32),
                   jax.ShapeDtypeStruct((NPAD, C), jnp.float32)),
    )(p, dinv, w, acc, w0, b2)


def _final_body(p_ref, dinv_ref, w_ref, acc_ref, out_ref):
    xs = (p_ref[0] + p_ref[1]) * dinv_ref[...]
    z = acc_ref[...] + jnp.dot(xs, w_ref[...],
                               preferred_element_type=jnp.float32)
    m = jnp.max(z, axis=1, keepdims=True)
    s = jnp.log(jnp.sum(jnp.exp(z - m), axis=1, keepdims=True))
    out_ref[...] = z - m - s


def _tc_final(p, dinv, w, acc):
    return pl.pallas_call(
        _final_body,
        grid=(NPAD // RBLK,),
        in_specs=[
            pl.BlockSpec((NC, RBLK, C), lambda i: (0, i, 0)),
            pl.BlockSpec((RBLK, 1), lambda i: (i, 0)),
            pl.BlockSpec((C, C), lambda i: (0, 0)),
            pl.BlockSpec((RBLK, C), lambda i: (i, 0)),
        ],
        out_specs=pl.BlockSpec((RBLK, C), lambda i: (i, 0)),
        out_shape=jax.ShapeDtypeStruct((NPAD, C), jnp.float32),
    )(p, dinv, w, acc)


def kernel(x, edge_index, W1, b1, W2, b2):
    row = edge_index[0].astype(jnp.int32)
    col = edge_index[1].astype(jnp.int32)
    padlen = EPAD - E
    fill = N + (jnp.arange(padlen, dtype=jnp.int32) % (NPAD - N))

    def _shard(v):
        ch = v.reshape(NCH, NTILES, CHUNK).transpose(1, 0, 2)
        return ch.reshape(NTILES * NHALF, NCHH, CHUNK)

    rowp = _shard(jnp.concatenate([row, fill]))
    colp = _shard(jnp.concatenate([col, fill]))
    hp = jnp.pad(x, ((0, NPAD - N), (0, 0)))
    b1r = b1.reshape(1, C)
    b2r = b2.reshape(1, C)

    degp = _sc_deg(colp.reshape(NTILES, NCH, CHUNK))
    dinv, dinv2 = _tc_prep(degp)

    g, acc = _tc_init(hp, dinv, W1[0:C], b1r, relu=False)
    for k in (1, 2):
        p = _sc_hop(g, rowp, colp)
        g, acc = _tc_hopc(p, dinv, dinv2, W1[C * k:C * (k + 1)], acc)
    p = _sc_hop(g, rowp, colp)
    g, acc = _tc_bridge(p, dinv, W1[3 * C:4 * C], acc, W2[0:C], b2r)
    for k in (1, 2):
        p = _sc_hop(g, rowp, colp)
        g, acc = _tc_hopc(p, dinv, dinv2, W2[C * k:C * (k + 1)], acc)
    p = _sc_hop(g, rowp, colp)
    out = _tc_final(p, dinv, W2[3 * C:4 * C], acc)
    return out[:N]
